# baseline (device time: 381190 ns/iter reference)
import jax
import jax.numpy as jnp
from jax import lax
from jax.experimental import pallas as pl
from jax.experimental.pallas import tpu as pltpu

N_DEV = 4
M = 1024
T = 8
W = 1024
H = W // 2


def _rs_call(x, w):
    m_glob, k = x.shape
    n = w.shape[1]

    def body(x_ref, w_ref, y_ref, amax_ref,
             nbr, own, relay_snd, comb_snd, direct_r, relay_r, comb_r,
             snd_sems, rcv_direct, rcv_relay, rcv_comb):
        t = pl.program_id(0)
        my = lax.axis_index("i")
        dev = [lax.rem(my + 1, N_DEV), lax.rem(my + N_DEV - 1, N_DEV)]
        cols = [slice(0, H), slice(H, W)]

        @pl.when(t == 0)
        def _():
            bar = pltpu.get_barrier_semaphore()
            for s in range(2):
                pl.semaphore_signal(bar, inc=1, device_id=(dev[s],),
                                    device_id_type=pl.DeviceIdType.MESH)
            pl.semaphore_wait(bar, 2)

        def mk_direct(par, s, d):
            return pltpu.make_async_remote_copy(
                src_ref=nbr.at[s, :, cols[s]],
                dst_ref=direct_r.at[par, s],
                send_sem=snd_sems.at[0, s],
                recv_sem=rcv_direct.at[par, s],
                device_id=(d,), device_id_type=pl.DeviceIdType.MESH)

        def mk_relay(par_snd, par, s, d):
            return pltpu.make_async_remote_copy(
                src_ref=relay_snd.at[par_snd, s],
                dst_ref=relay_r.at[par, s],
                send_sem=snd_sems.at[1, s],
                recv_sem=rcv_relay.at[par, s],
                device_id=(d,), device_id_type=pl.DeviceIdType.MESH)

        def mk_comb(par_snd, par, s, d):
            return pltpu.make_async_remote_copy(
                src_ref=comb_snd.at[par_snd, s],
                dst_ref=comb_r.at[par, s],
                send_sem=snd_sems.at[2, s],
                recv_sem=rcv_comb.at[par, s],
                device_id=(d,), device_id_type=pl.DeviceIdType.MESH)

        @pl.when(jnp.logical_and(t >= 1, t <= T))
        def _():
            for s in range(2):
                mk_direct(0, s, my).wait_send()
                mk_relay(lax.rem(t - 1, 2), 0, s, my).wait_send()

        @pl.when(jnp.logical_and(t >= 2, t <= T + 1))
        def _():
            for s in range(2):
                mk_comb(lax.rem(t - 2, 2), 0, s, my).wait_send()

        @pl.when(jnp.logical_and(t >= 1, t <= T))
        def _():
            pr = lax.rem(t - 1, 3)
            pc = lax.rem(t - 1, 2)
            p4 = lax.rem(t - 1, 4)
            for s in range(2):
                mk_relay(0, pr, s, my).wait_recv()
                comb_snd[pc, s] = (
                    nbr[1 - s, :, cols[s]].astype(jnp.float32)
                    + relay_r[pr, s].astype(jnp.float32)
                ).astype(jnp.bfloat16)
                mk_comb(pc, p4, s, dev[1 - s]).start()

        @pl.when(t <= T - 1)
        def _():
            p4 = lax.rem(t, 4)
            p3 = lax.rem(t, 3)
            p2 = lax.rem(t, 2)
            nbr[0] = jnp.dot(
                x_ref[pl.ds(dev[0] * M, M), :], w_ref[...],
                preferred_element_type=jnp.float32).astype(jnp.bfloat16)
            nbr[1] = jnp.dot(
                x_ref[pl.ds(dev[1] * M, M), :], w_ref[...],
                preferred_element_type=jnp.float32).astype(jnp.bfloat16)
            diag = lax.rem(my + 2, N_DEV)
            for s in range(2):
                mk_direct(p4, s, dev[s]).start()
                relay_snd[p2, s] = jnp.dot(
                    x_ref[pl.ds(diag * M, M), :], w_ref[:, cols[s]],
                    preferred_element_type=jnp.float32).astype(jnp.bfloat16)
                mk_relay(p2, p3, s, dev[1 - s]).start()

        @pl.when(t >= 2)
        def _():
            q = lax.rem(t - 2, 4)
            po = lax.rem(t - 2, 2)
            tmax = jnp.float32(0)
            for s in range(2):
                mk_direct(q, s, my).wait_recv()
                mk_comb(0, q, s, my).wait_recv()
                acc = (own[po, :, cols[s]].astype(jnp.float32)
                       + direct_r[q, s].astype(jnp.float32)
                       + comb_r[q, s].astype(jnp.float32))
                ys = jnp.maximum(acc, 0.0)
                y_ref[:, cols[s]] = ys
                tmax = jnp.maximum(tmax, jnp.max(ys))

            @pl.when(t == 2)
            def _():
                amax_ref[...] = jnp.full((1, 128), tmax, jnp.float32)

            @pl.when(t > 2)
            def _():
                amax_ref[...] = jnp.maximum(amax_ref[...], tmax)

        @pl.when(t <= T - 1)
        def _():
            own[lax.rem(t, 2)] = jnp.dot(
                x_ref[pl.ds(my * M, M), :], w_ref[...],
                preferred_element_type=jnp.float32).astype(jnp.bfloat16)

    return pl.pallas_call(
        body,
        grid=(T + 2,),
        out_shape=[
            jax.ShapeDtypeStruct((M, n), jnp.float32),
            jax.ShapeDtypeStruct((1, 128), jnp.float32),
        ],
        in_specs=[
            pl.BlockSpec((m_glob, k), lambda t: (0, 0),
                         memory_space=pltpu.VMEM),
            pl.BlockSpec((k, W), lambda t: (0, jnp.minimum(t, T - 1)),
                         memory_space=pltpu.VMEM),
        ],
        out_specs=[
            pl.BlockSpec((M, W), lambda t: (0, jnp.maximum(t - 2, 0)),
                         memory_space=pltpu.VMEM),
            pl.BlockSpec((1, 128), lambda t: (0, 0),
                         memory_space=pltpu.VMEM),
        ],
        scratch_shapes=[
            pltpu.VMEM((2, M, W), jnp.bfloat16),
            pltpu.VMEM((2, M, W), jnp.bfloat16),
            pltpu.VMEM((2, 2, M, H), jnp.bfloat16),
            pltpu.VMEM((2, 2, M, H), jnp.bfloat16),
            pltpu.VMEM((4, 2, M, H), jnp.bfloat16),
            pltpu.VMEM((3, 2, M, H), jnp.bfloat16),
            pltpu.VMEM((4, 2, M, H), jnp.bfloat16),
            pltpu.SemaphoreType.DMA((3, 2)),
            pltpu.SemaphoreType.DMA((4, 2)),
            pltpu.SemaphoreType.DMA((3, 2)),
            pltpu.SemaphoreType.DMA((4, 2)),
        ],
        compiler_params=pltpu.CompilerParams(
            collective_id=0, dimension_semantics=("arbitrary",),
            vmem_limit_bytes=63 * 1024 * 1024,
        ),
    )(x, w)


def _quant_call(y, amax_local):
    m, n = y.shape
    t2 = 4
    w2 = n // t2

    def body(y_ref, amax_ref, out_ref, exch, send_sems, recv_sems):
        t = pl.program_id(0)
        my = lax.axis_index("i")

        @pl.when(t == 0)
        def _():
            exch[N_DEV - 1, :] = amax_ref[0, :]
            rdmas = []
            for r in range(1, N_DEV):
                rd = pltpu.make_async_remote_copy(
                    src_ref=exch.at[N_DEV - 1],
                    dst_ref=exch.at[N_DEV - 1 - r],
                    send_sem=send_sems.at[r - 1],
                    recv_sem=recv_sems.at[N_DEV - 1 - r],
                    device_id=(lax.rem(my + r, N_DEV),),
                    device_id_type=pl.DeviceIdType.MESH,
                )
                rd.start()
                rdmas.append(rd)
            for rd in rdmas:
                rd.wait_send()
                rd.wait_recv()

        g = jnp.max(exch[:, 0])
        inv = 448.0 / g
        scale = g * (1.0 / 448.0)
        yv = y_ref[...]
        q = jnp.minimum(yv * inv, 448.0).astype(jnp.float8_e4m3fn)
        out_ref[...] = q.astype(jnp.float32) * scale

    return pl.pallas_call(
        body,
        grid=(t2,),
        out_shape=jax.ShapeDtypeStruct((m, n), jnp.float32),
        in_specs=[
            pl.BlockSpec((m, w2), lambda t: (0, t), memory_space=pltpu.VMEM),
            pl.BlockSpec((1, 128), lambda t: (0, 0),
                         memory_space=pltpu.VMEM),
        ],
        out_specs=pl.BlockSpec((m, w2), lambda t: (0, t),
                               memory_space=pltpu.VMEM),
        input_output_aliases={0: 0},
        scratch_shapes=[
            pltpu.VMEM((N_DEV, 128), jnp.float32),
            pltpu.SemaphoreType.DMA((N_DEV - 1,)),
            pltpu.SemaphoreType.DMA((N_DEV - 1,)),
        ],
        compiler_params=pltpu.CompilerParams(
            dimension_semantics=("arbitrary",),
            vmem_limit_bytes=63 * 1024 * 1024,
        ),
    )(y, amax_local)


def kernel(x, w_mat):
    xb = x.astype(jnp.bfloat16)
    wb = w_mat.astype(jnp.bfloat16)
    y, amax_local = _rs_call(xb, wb)
    return _quant_call(y, amax_local)


# device time: 368808 ns/iter; 1.0336x vs baseline; 1.0336x over previous
import jax
import jax.numpy as jnp
from jax import lax
from jax.experimental import pallas as pl
from jax.experimental.pallas import tpu as pltpu

N_DEV = 4
M = 1024
T = 8
W = 1024
H = W // 2


def _rs_call(x, w):
    m_glob, k = x.shape
    n = w.shape[1]

    def body(x_ref, w_ref, y_ref, amax_ref,
             nbr, own, relay_snd, comb_snd, direct_r, relay_r, comb_r,
             snd_sems, rcv_direct, rcv_relay, rcv_comb):
        t = pl.program_id(0)
        my = lax.axis_index("i")
        dev = [lax.rem(my + 1, N_DEV), lax.rem(my + N_DEV - 1, N_DEV)]
        cols = [slice(0, H), slice(H, W)]

        @pl.when(t == 0)
        def _():
            bar = pltpu.get_barrier_semaphore()
            for s in range(2):
                pl.semaphore_signal(bar, inc=1, device_id=(dev[s],),
                                    device_id_type=pl.DeviceIdType.MESH)
            pl.semaphore_wait(bar, 2)

        def mk_direct(par, s, d):
            return pltpu.make_async_remote_copy(
                src_ref=nbr.at[s, :, cols[s]],
                dst_ref=direct_r.at[par, s],
                send_sem=snd_sems.at[0, s],
                recv_sem=rcv_direct.at[par, s],
                device_id=(d,), device_id_type=pl.DeviceIdType.MESH)

        def mk_relay(par_snd, par, s, d):
            return pltpu.make_async_remote_copy(
                src_ref=relay_snd.at[par_snd, s],
                dst_ref=relay_r.at[par, s],
                send_sem=snd_sems.at[1, s],
                recv_sem=rcv_relay.at[par, s],
                device_id=(d,), device_id_type=pl.DeviceIdType.MESH)

        def mk_comb(par_snd, par, s, d):
            return pltpu.make_async_remote_copy(
                src_ref=comb_snd.at[par_snd, s],
                dst_ref=comb_r.at[par, s],
                send_sem=snd_sems.at[2, s],
                recv_sem=rcv_comb.at[par, s],
                device_id=(d,), device_id_type=pl.DeviceIdType.MESH)

        @pl.when(jnp.logical_and(t >= 1, t <= T))
        def _():
            for s in range(2):
                mk_direct(0, s, my).wait_send()
                mk_relay(lax.rem(t - 1, 2), 0, s, my).wait_send()

        @pl.when(jnp.logical_and(t >= 2, t <= T + 1))
        def _():
            for s in range(2):
                mk_comb(lax.rem(t - 2, 2), 0, s, my).wait_send()

        @pl.when(jnp.logical_and(t >= 1, t <= T))
        def _():
            pr = lax.rem(t - 1, 3)
            pc = lax.rem(t - 1, 2)
            p4 = lax.rem(t - 1, 4)
            for s in range(2):
                mk_relay(0, pr, s, my).wait_recv()
                comb_snd[pc, s] = (
                    nbr[1 - s, :, cols[s]].astype(jnp.float32)
                    + relay_r[pr, s].astype(jnp.float32)
                ).astype(jnp.bfloat16)
                mk_comb(pc, p4, s, dev[1 - s]).start()

        @pl.when(t <= T - 1)
        def _():
            p4 = lax.rem(t, 4)
            p3 = lax.rem(t, 3)
            p2 = lax.rem(t, 2)
            nbr[0] = jnp.dot(
                x_ref[pl.ds(dev[0] * M, M), :], w_ref[...],
                preferred_element_type=jnp.float32).astype(jnp.bfloat16)
            nbr[1] = jnp.dot(
                x_ref[pl.ds(dev[1] * M, M), :], w_ref[...],
                preferred_element_type=jnp.float32).astype(jnp.bfloat16)
            diag = lax.rem(my + 2, N_DEV)
            for s in range(2):
                mk_direct(p4, s, dev[s]).start()
                relay_snd[p2, s] = jnp.dot(
                    x_ref[pl.ds(diag * M, M), :], w_ref[:, cols[s]],
                    preferred_element_type=jnp.float32).astype(jnp.bfloat16)
                mk_relay(p2, p3, s, dev[1 - s]).start()

        @pl.when(t >= 2)
        def _():
            q = lax.rem(t - 2, 4)
            po = lax.rem(t - 2, 2)
            tmax = jnp.float32(0)
            for s in range(2):
                mk_direct(q, s, my).wait_recv()
                mk_comb(0, q, s, my).wait_recv()
                acc = (own[po, :, cols[s]].astype(jnp.float32)
                       + direct_r[q, s].astype(jnp.float32)
                       + comb_r[q, s].astype(jnp.float32))
                ys = jnp.maximum(acc, 0.0)
                y_ref[:, cols[s]] = ys
                tmax = jnp.maximum(tmax, jnp.max(ys))

            @pl.when(t == 2)
            def _():
                amax_ref[...] = jnp.full((1, 128), tmax, jnp.float32)

            @pl.when(t > 2)
            def _():
                amax_ref[...] = jnp.maximum(amax_ref[...], tmax)

        @pl.when(t <= T - 1)
        def _():
            own[lax.rem(t, 2)] = jnp.dot(
                x_ref[pl.ds(my * M, M), :], w_ref[...],
                preferred_element_type=jnp.float32).astype(jnp.bfloat16)

    return pl.pallas_call(
        body,
        grid=(T + 2,),
        out_shape=[
            jax.ShapeDtypeStruct((M, n), jnp.float32),
            jax.ShapeDtypeStruct((1, 128), jnp.float32),
        ],
        in_specs=[
            pl.BlockSpec((m_glob, k), lambda t: (0, 0),
                         memory_space=pltpu.VMEM),
            pl.BlockSpec((k, W), lambda t: (0, jnp.minimum(t, T - 1)),
                         memory_space=pltpu.VMEM),
        ],
        out_specs=[
            pl.BlockSpec((M, W), lambda t: (0, jnp.maximum(t - 2, 0)),
                         memory_space=pltpu.VMEM),
            pl.BlockSpec((1, 128), lambda t: (0, 0),
                         memory_space=pltpu.VMEM),
        ],
        scratch_shapes=[
            pltpu.VMEM((2, M, W), jnp.bfloat16),
            pltpu.VMEM((2, M, W), jnp.bfloat16),
            pltpu.VMEM((2, 2, M, H), jnp.bfloat16),
            pltpu.VMEM((2, 2, M, H), jnp.bfloat16),
            pltpu.VMEM((4, 2, M, H), jnp.bfloat16),
            pltpu.VMEM((3, 2, M, H), jnp.bfloat16),
            pltpu.VMEM((4, 2, M, H), jnp.bfloat16),
            pltpu.SemaphoreType.DMA((3, 2)),
            pltpu.SemaphoreType.DMA((4, 2)),
            pltpu.SemaphoreType.DMA((3, 2)),
            pltpu.SemaphoreType.DMA((4, 2)),
        ],
        compiler_params=pltpu.CompilerParams(
            collective_id=0, dimension_semantics=("arbitrary",),
            vmem_limit_bytes=63 * 1024 * 1024,
        ),
    )(x, w)


def _quant_call(y, amax_local):
    m, n = y.shape
    t2 = 4
    w2 = n // t2

    def body(y_ref, amax_ref, out_ref, exch, send_sems, recv_sems):
        t = pl.program_id(0)
        my = lax.axis_index("i")

        @pl.when(t == 0)
        def _():
            exch[N_DEV - 1, :] = amax_ref[0, :]
            rdmas = []
            for r in range(1, N_DEV):
                rd = pltpu.make_async_remote_copy(
                    src_ref=exch.at[N_DEV - 1],
                    dst_ref=exch.at[N_DEV - 1 - r],
                    send_sem=send_sems.at[r - 1],
                    recv_sem=recv_sems.at[N_DEV - 1 - r],
                    device_id=(lax.rem(my + r, N_DEV),),
                    device_id_type=pl.DeviceIdType.MESH,
                )
                rd.start()
                rdmas.append(rd)
            for rd in rdmas:
                rd.wait_send()
                rd.wait_recv()

        g = jnp.max(exch[:, 0])
        inv = 448.0 / g
        scale = g * (1.0 / 448.0)
        yv = y_ref[...]
        q = jnp.minimum(yv * inv, 448.0).astype(jnp.float8_e4m3fn)
        out_ref[...] = (q.astype(jnp.float32) * scale).astype(jnp.bfloat16)

    return pl.pallas_call(
        body,
        grid=(t2,),
        out_shape=jax.ShapeDtypeStruct((m, n), jnp.bfloat16),
        in_specs=[
            pl.BlockSpec((m, w2), lambda t: (0, t), memory_space=pltpu.VMEM),
            pl.BlockSpec((1, 128), lambda t: (0, 0),
                         memory_space=pltpu.VMEM),
        ],
        out_specs=pl.BlockSpec((m, w2), lambda t: (0, t),
                               memory_space=pltpu.VMEM),
        scratch_shapes=[
            pltpu.VMEM((N_DEV, 128), jnp.float32),
            pltpu.SemaphoreType.DMA((N_DEV - 1,)),
            pltpu.SemaphoreType.DMA((N_DEV - 1,)),
        ],
        compiler_params=pltpu.CompilerParams(
            dimension_semantics=("arbitrary",),
            vmem_limit_bytes=63 * 1024 * 1024,
        ),
    )(y, amax_local)


def kernel(x, w_mat):
    xb = x.astype(jnp.bfloat16)
    wb = w_mat.astype(jnp.bfloat16)
    y, amax_local = _rs_call(xb, wb)
    return _quant_call(y, amax_local)


# device time: 351311 ns/iter; 1.0850x vs baseline; 1.0498x over previous
import jax
import jax.numpy as jnp
from jax import lax
from jax.experimental import pallas as pl
from jax.experimental.pallas import tpu as pltpu

N_DEV = 4
M = 1024
T = 8
W = 1024
H = W // 2


def _rs_call(x, w):
    m_glob, k = x.shape
    n = w.shape[1]

    def body(x_ref, w_ref, y_ref, amax_ref,
             nbr, own, relay_snd, comb_snd, direct_r, relay_r, comb_r,
             snd_sems, rcv_direct, rcv_relay, rcv_comb):
        t = pl.program_id(0)
        my = lax.axis_index("i")
        dev = [lax.rem(my + 1, N_DEV), lax.rem(my + N_DEV - 1, N_DEV)]
        cols = [slice(0, H), slice(H, W)]

        @pl.when(t == 0)
        def _():
            bar = pltpu.get_barrier_semaphore()
            for s in range(2):
                pl.semaphore_signal(bar, inc=1, device_id=(dev[s],),
                                    device_id_type=pl.DeviceIdType.MESH)
            pl.semaphore_wait(bar, 2)

        def mk_direct(par, s, d):
            return pltpu.make_async_remote_copy(
                src_ref=nbr.at[s, :, cols[s]],
                dst_ref=direct_r.at[par, s],
                send_sem=snd_sems.at[0, s],
                recv_sem=rcv_direct.at[par, s],
                device_id=(d,), device_id_type=pl.DeviceIdType.MESH)

        def mk_relay(par, s, d):
            return pltpu.make_async_remote_copy(
                src_ref=relay_snd.at[s],
                dst_ref=relay_r.at[par, s],
                send_sem=snd_sems.at[1, s],
                recv_sem=rcv_relay.at[par, s],
                device_id=(d,), device_id_type=pl.DeviceIdType.MESH)

        def mk_comb(par, s, d):
            return pltpu.make_async_remote_copy(
                src_ref=comb_snd.at[s],
                dst_ref=comb_r.at[par, s],
                send_sem=snd_sems.at[2, s],
                recv_sem=rcv_comb.at[par, s],
                device_id=(d,), device_id_type=pl.DeviceIdType.MESH)

        @pl.when(jnp.logical_and(t >= 1, t <= T))
        def _():
            for s in range(2):
                mk_direct(0, s, my).wait_send()
                mk_relay(0, s, my).wait_send()

        @pl.when(jnp.logical_and(t >= 2, t <= T + 1))
        def _():
            for s in range(2):
                mk_comb(0, s, my).wait_send()

        @pl.when(jnp.logical_and(t >= 1, t <= T))
        def _():
            pr = lax.rem(t - 1, 3)
            p4 = lax.rem(t - 1, 4)
            for s in range(2):
                mk_relay(pr, s, my).wait_recv()
                comb_snd[s] = (
                    nbr[1 - s, :, cols[s]].astype(jnp.float32)
                    + relay_r[pr, s].astype(jnp.float32)
                ).astype(jnp.bfloat16)
                mk_comb(p4, s, dev[1 - s]).start()

        @pl.when(t <= T - 1)
        def _():
            p4 = lax.rem(t, 4)
            p3 = lax.rem(t, 3)
            wv = w_ref[...].astype(jnp.bfloat16)
            nbr[0] = jnp.dot(
                x_ref[pl.ds(dev[0] * M, M), :], wv,
                preferred_element_type=jnp.float32).astype(jnp.bfloat16)
            nbr[1] = jnp.dot(
                x_ref[pl.ds(dev[1] * M, M), :], wv,
                preferred_element_type=jnp.float32).astype(jnp.bfloat16)
            diag = lax.rem(my + 2, N_DEV)
            for s in range(2):
                mk_direct(p4, s, dev[s]).start()
                relay_snd[s] = jnp.dot(
                    x_ref[pl.ds(diag * M, M), :], wv[:, cols[s]],
                    preferred_element_type=jnp.float32).astype(jnp.bfloat16)
                mk_relay(p3, s, dev[1 - s]).start()

        @pl.when(t >= 2)
        def _():
            q = lax.rem(t - 2, 4)
            po = lax.rem(t - 2, 2)
            tmax = jnp.float32(0)
            for s in range(2):
                mk_direct(q, s, my).wait_recv()
                mk_comb(q, s, my).wait_recv()
                acc = (own[po, :, cols[s]].astype(jnp.float32)
                       + direct_r[q, s].astype(jnp.float32)
                       + comb_r[q, s].astype(jnp.float32))
                ys = jnp.maximum(acc, 0.0)
                y_ref[:, cols[s]] = ys
                tmax = jnp.maximum(tmax, jnp.max(ys))

            @pl.when(t == 2)
            def _():
                amax_ref[...] = jnp.full((1, 128), tmax, jnp.float32)

            @pl.when(t > 2)
            def _():
                amax_ref[...] = jnp.maximum(amax_ref[...], tmax)

        @pl.when(t <= T - 1)
        def _():
            own[lax.rem(t, 2)] = jnp.dot(
                x_ref[pl.ds(my * M, M), :], w_ref[...].astype(jnp.bfloat16),
                preferred_element_type=jnp.float32).astype(jnp.bfloat16)

    return pl.pallas_call(
        body,
        grid=(T + 2,),
        out_shape=[
            jax.ShapeDtypeStruct((M, n), jnp.float32),
            jax.ShapeDtypeStruct((1, 128), jnp.float32),
        ],
        in_specs=[
            pl.BlockSpec((m_glob, k), lambda t: (0, 0),
                         memory_space=pltpu.VMEM),
            pl.BlockSpec((k, W), lambda t: (0, jnp.minimum(t, T - 1)),
                         memory_space=pltpu.VMEM),
        ],
        out_specs=[
            pl.BlockSpec((M, W), lambda t: (0, jnp.maximum(t - 2, 0)),
                         memory_space=pltpu.VMEM),
            pl.BlockSpec((1, 128), lambda t: (0, 0),
                         memory_space=pltpu.VMEM),
        ],
        scratch_shapes=[
            pltpu.VMEM((2, M, W), jnp.bfloat16),
            pltpu.VMEM((2, M, W), jnp.bfloat16),
            pltpu.VMEM((2, M, H), jnp.bfloat16),
            pltpu.VMEM((2, M, H), jnp.bfloat16),
            pltpu.VMEM((4, 2, M, H), jnp.bfloat16),
            pltpu.VMEM((3, 2, M, H), jnp.bfloat16),
            pltpu.VMEM((4, 2, M, H), jnp.bfloat16),
            pltpu.SemaphoreType.DMA((3, 2)),
            pltpu.SemaphoreType.DMA((4, 2)),
            pltpu.SemaphoreType.DMA((3, 2)),
            pltpu.SemaphoreType.DMA((4, 2)),
        ],
        compiler_params=pltpu.CompilerParams(
            collective_id=0, dimension_semantics=("arbitrary",),
            vmem_limit_bytes=63 * 1024 * 1024,
        ),
    )(x, w)


def _quant_call(y, amax_local):
    m, n = y.shape
    t2 = 4
    w2 = n // t2

    def body(y_ref, amax_ref, out_ref, exch, send_sems, recv_sems):
        t = pl.program_id(0)
        my = lax.axis_index("i")

        @pl.when(t == 0)
        def _():
            exch[N_DEV - 1, :] = amax_ref[0, :]
            rdmas = []
            for r in range(1, N_DEV):
                rd = pltpu.make_async_remote_copy(
                    src_ref=exch.at[N_DEV - 1],
                    dst_ref=exch.at[N_DEV - 1 - r],
                    send_sem=send_sems.at[r - 1],
                    recv_sem=recv_sems.at[N_DEV - 1 - r],
                    device_id=(lax.rem(my + r, N_DEV),),
                    device_id_type=pl.DeviceIdType.MESH,
                )
                rd.start()
                rdmas.append(rd)
            for rd in rdmas:
                rd.wait_send()
                rd.wait_recv()

        g = jnp.max(exch[:, 0])
        inv = 448.0 / g
        scale = g * (1.0 / 448.0)
        yv = y_ref[...]
        q = jnp.minimum(yv * inv, 448.0).astype(jnp.float8_e4m3fn)
        out_ref[...] = (q.astype(jnp.float32) * scale).astype(jnp.bfloat16)

    return pl.pallas_call(
        body,
        grid=(t2,),
        out_shape=jax.ShapeDtypeStruct((m, n), jnp.bfloat16),
        in_specs=[
            pl.BlockSpec((m, w2), lambda t: (0, t), memory_space=pltpu.VMEM),
            pl.BlockSpec((1, 128), lambda t: (0, 0),
                         memory_space=pltpu.VMEM),
        ],
        out_specs=pl.BlockSpec((m, w2), lambda t: (0, t),
                               memory_space=pltpu.VMEM),
        scratch_shapes=[
            pltpu.VMEM((N_DEV, 128), jnp.float32),
            pltpu.SemaphoreType.DMA((N_DEV - 1,)),
            pltpu.SemaphoreType.DMA((N_DEV - 1,)),
        ],
        compiler_params=pltpu.CompilerParams(
            dimension_semantics=("arbitrary",),
            vmem_limit_bytes=63 * 1024 * 1024,
        ),
    )(y, amax_local)


def kernel(x, w_mat):
    xb = x.astype(jnp.bfloat16)
    y, amax_local = _rs_call(xb, w_mat)
    return _quant_call(y, amax_local)
